# baseline (device time: 9751 ns/iter reference)
import jax
import jax.numpy as jnp
from jax import lax
from jax.experimental import pallas as pl
from jax.experimental.pallas import tpu as pltpu

N_DEV = 4
N_TOK = 256
D_IN = 128
D_OUT = 256
N_EXP = 8
E_LOCAL = N_EXP // N_DEV
ROWS = N_TOK // N_DEV


def kernel(x, router_W, route_idx, expert_W):
    def body(x_ref, rw_ref, idx_ref, ew_ref, out_ref,
             partial_ref, recv_ref, send_sems, recv_sems):
        my_i = lax.axis_index("i")

        xv = x_ref[:, :]
        scores = jnp.dot(xv, rw_ref[:, :], preferred_element_type=jnp.float32)
        s_max = jnp.max(scores, axis=-1, keepdims=True)
        es = jnp.exp(scores - s_max)
        idx0 = idx_ref[:, 0:1]
        idx1 = idx_ref[:, 1:2]
        iota = lax.broadcasted_iota(jnp.int32, (N_TOK, N_EXP), 1)
        g0 = jnp.sum(jnp.where(iota == idx0, es, 0.0), axis=1, keepdims=True)
        g1 = jnp.sum(jnp.where(iota == idx1, es, 0.0), axis=1, keepdims=True)
        gs = g0 + g1
        w0 = g0 / gs
        w1 = g1 / gs
        acc = jnp.zeros((N_TOK, D_OUT), jnp.float32)
        for le in range(E_LOCAL):
            e = my_i * E_LOCAL + le
            w = jnp.where(idx0 == e, w0, 0.0) + jnp.where(idx1 == e, w1, 0.0)
            acc = acc + jnp.dot(xv * w, ew_ref[le],
                                preferred_element_type=jnp.float32)
        partial_ref[:, :] = acc

        barrier_sem = pltpu.get_barrier_semaphore()
        for d in range(N_DEV):
            @pl.when(my_i != d)
            def _():
                pl.semaphore_signal(barrier_sem, inc=1, device_id=(d,),
                                    device_id_type=pl.DeviceIdType.MESH)
        pl.semaphore_wait(barrier_sem, N_DEV - 1)

        for d in range(N_DEV):
            @pl.when(my_i != d)
            def _():
                rdma = pltpu.make_async_remote_copy(
                    src_ref=partial_ref.at[pl.ds(d * ROWS, ROWS), :],
                    dst_ref=recv_ref.at[my_i],
                    send_sem=send_sems.at[d],
                    recv_sem=recv_sems.at[my_i],
                    device_id=(d,),
                    device_id_type=pl.DeviceIdType.MESH,
                )
                rdma.start()

        out_ref[:, :] = partial_ref[pl.ds(my_i * ROWS, ROWS), :]

        for j in range(N_DEV):
            @pl.when(my_i != j)
            def _():
                recv = pltpu.make_async_remote_copy(
                    src_ref=partial_ref.at[pl.ds(0, ROWS), :],
                    dst_ref=recv_ref.at[j],
                    send_sem=send_sems.at[j],
                    recv_sem=recv_sems.at[j],
                    device_id=(j,),
                    device_id_type=pl.DeviceIdType.MESH,
                )
                recv.wait_recv()
                out_ref[:, :] = out_ref[:, :] + recv_ref[j]

        for d in range(N_DEV):
            @pl.when(my_i != d)
            def _():
                send = pltpu.make_async_remote_copy(
                    src_ref=partial_ref.at[pl.ds(d * ROWS, ROWS), :],
                    dst_ref=recv_ref.at[0],
                    send_sem=send_sems.at[d],
                    recv_sem=recv_sems.at[d],
                    device_id=(d,),
                    device_id_type=pl.DeviceIdType.MESH,
                )
                send.wait_send()

    return pl.pallas_call(
        body,
        out_shape=jax.ShapeDtypeStruct((ROWS, D_OUT), jnp.float32),
        in_specs=[
            pl.BlockSpec(memory_space=pltpu.VMEM),
            pl.BlockSpec(memory_space=pltpu.VMEM),
            pl.BlockSpec(memory_space=pltpu.VMEM),
            pl.BlockSpec(memory_space=pltpu.VMEM),
        ],
        out_specs=pl.BlockSpec(memory_space=pltpu.VMEM),
        scratch_shapes=[
            pltpu.VMEM((N_TOK, D_OUT), jnp.float32),
            pltpu.VMEM((N_DEV, ROWS, D_OUT), jnp.float32),
            pltpu.SemaphoreType.DMA((N_DEV,)),
            pltpu.SemaphoreType.DMA((N_DEV,)),
        ],
        compiler_params=pltpu.CompilerParams(collective_id=0),
    )(x, router_W, route_idx, expert_W)


# device time: 9055 ns/iter; 1.0769x vs baseline; 1.0769x over previous
import jax
import jax.numpy as jnp
from jax import lax
from jax.experimental import pallas as pl
from jax.experimental.pallas import tpu as pltpu

N_DEV = 4
N_TOK = 256
D_IN = 128
D_OUT = 256
N_EXP = 8
E_LOCAL = N_EXP // N_DEV
ROWS = N_TOK // N_DEV


def kernel(x, router_W, route_idx, expert_W):
    def body(x_ref, rw_ref, idx_ref, ew_ref, out_ref,
             partial_ref, recv_ref, send_sems, recv_sems):
        my_i = lax.axis_index("i")

        barrier_sem = pltpu.get_barrier_semaphore()
        for d in range(N_DEV):
            @pl.when(my_i != d)
            def _():
                pl.semaphore_signal(barrier_sem, inc=1, device_id=(d,),
                                    device_id_type=pl.DeviceIdType.MESH)

        xv = x_ref[:, :]
        scores = jnp.dot(xv, rw_ref[:, :], preferred_element_type=jnp.float32)
        s_max = jnp.max(scores, axis=-1, keepdims=True)
        es = jnp.exp(scores - s_max)
        idx0 = idx_ref[:, 0:1]
        idx1 = idx_ref[:, 1:2]
        iota = lax.broadcasted_iota(jnp.int32, (N_TOK, N_EXP), 1)
        g0 = jnp.sum(jnp.where(iota == idx0, es, 0.0), axis=1, keepdims=True)
        g1 = jnp.sum(jnp.where(iota == idx1, es, 0.0), axis=1, keepdims=True)
        gs = g0 + g1
        w0 = g0 / gs
        w1 = g1 / gs

        e0 = my_i * E_LOCAL
        e1 = e0 + 1
        we0 = jnp.where(idx0 == e0, w0, 0.0) + jnp.where(idx1 == e0, w1, 0.0)
        we1 = jnp.where(idx0 == e1, w0, 0.0) + jnp.where(idx1 == e1, w1, 0.0)
        x2 = jnp.concatenate([xv * we0, xv * we1], axis=1)
        w2 = jnp.concatenate([ew_ref[0], ew_ref[1]], axis=0)

        pl.semaphore_wait(barrier_sem, N_DEV - 1)

        for d in range(N_DEV):
            chunk = jnp.dot(x2[d * ROWS:(d + 1) * ROWS, :], w2,
                            preferred_element_type=jnp.float32)
            partial_ref[pl.ds(d * ROWS, ROWS), :] = chunk

            @pl.when(my_i == d)
            def _():
                out_ref[:, :] = chunk

            @pl.when(my_i != d)
            def _():
                rdma = pltpu.make_async_remote_copy(
                    src_ref=partial_ref.at[pl.ds(d * ROWS, ROWS), :],
                    dst_ref=recv_ref.at[my_i],
                    send_sem=send_sems.at[d],
                    recv_sem=recv_sems.at[my_i],
                    device_id=(d,),
                    device_id_type=pl.DeviceIdType.MESH,
                )
                rdma.start()

        for j in range(N_DEV):
            @pl.when(my_i != j)
            def _():
                recv = pltpu.make_async_remote_copy(
                    src_ref=partial_ref.at[pl.ds(0, ROWS), :],
                    dst_ref=recv_ref.at[j],
                    send_sem=send_sems.at[j],
                    recv_sem=recv_sems.at[j],
                    device_id=(j,),
                    device_id_type=pl.DeviceIdType.MESH,
                )
                recv.wait_recv()
                out_ref[:, :] = out_ref[:, :] + recv_ref[j]

        for d in range(N_DEV):
            @pl.when(my_i != d)
            def _():
                send = pltpu.make_async_remote_copy(
                    src_ref=partial_ref.at[pl.ds(d * ROWS, ROWS), :],
                    dst_ref=recv_ref.at[0],
                    send_sem=send_sems.at[d],
                    recv_sem=recv_sems.at[d],
                    device_id=(d,),
                    device_id_type=pl.DeviceIdType.MESH,
                )
                send.wait_send()

    return pl.pallas_call(
        body,
        out_shape=jax.ShapeDtypeStruct((ROWS, D_OUT), jnp.float32),
        in_specs=[
            pl.BlockSpec(memory_space=pltpu.VMEM),
            pl.BlockSpec(memory_space=pltpu.VMEM),
            pl.BlockSpec(memory_space=pltpu.VMEM),
            pl.BlockSpec(memory_space=pltpu.VMEM),
        ],
        out_specs=pl.BlockSpec(memory_space=pltpu.VMEM),
        scratch_shapes=[
            pltpu.VMEM((N_TOK, D_OUT), jnp.float32),
            pltpu.VMEM((N_DEV, ROWS, D_OUT), jnp.float32),
            pltpu.SemaphoreType.DMA((N_DEV,)),
            pltpu.SemaphoreType.DMA((N_DEV,)),
        ],
        compiler_params=pltpu.CompilerParams(collective_id=0),
    )(x, router_W, route_idx, expert_W)


# device time: 8533 ns/iter; 1.1427x vs baseline; 1.0612x over previous
import jax
import jax.numpy as jnp
from jax import lax
from jax.experimental import pallas as pl
from jax.experimental.pallas import tpu as pltpu

N_DEV = 4
N_TOK = 256
D_IN = 128
D_OUT = 256
N_EXP = 8
E_LOCAL = N_EXP // N_DEV
ROWS = N_TOK // N_DEV


def kernel(x, router_W, route_idx, expert_W):
    def body(x_ref, rw_ref, idx_ref, ew_ref, out_ref,
             partial_ref, recv_ref, send_sems, recv_sems):
        my_i = lax.axis_index("i")

        barrier_sem = pltpu.get_barrier_semaphore()
        for d in range(N_DEV):
            @pl.when(my_i != d)
            def _():
                pl.semaphore_signal(barrier_sem, inc=1, device_id=(d,),
                                    device_id_type=pl.DeviceIdType.MESH)

        xv = x_ref[:, :]
        scores = jnp.dot(xv, rw_ref[:, :], preferred_element_type=jnp.float32)
        idx0 = idx_ref[:, 0:1]
        idx1 = idx_ref[:, 1:2]
        iota = lax.broadcasted_iota(jnp.int32, (N_TOK, N_EXP), 1)
        s0 = jnp.sum(jnp.where(iota == idx0, scores, 0.0), axis=1, keepdims=True)
        s1 = jnp.sum(jnp.where(iota == idx1, scores, 0.0), axis=1, keepdims=True)
        w0 = 1.0 / (1.0 + jnp.exp(s1 - s0))
        w1 = 1.0 - w0

        e0 = my_i * E_LOCAL
        e1 = e0 + 1
        we0 = jnp.where(idx0 == e0, w0, 0.0) + jnp.where(idx1 == e0, w1, 0.0)
        we1 = jnp.where(idx0 == e1, w0, 0.0) + jnp.where(idx1 == e1, w1, 0.0)
        x2 = jnp.concatenate([xv * we0, xv * we1], axis=1).astype(jnp.bfloat16)
        w2 = jnp.concatenate([ew_ref[0], ew_ref[1]], axis=0).astype(jnp.bfloat16)

        pl.semaphore_wait(barrier_sem, N_DEV - 1)

        for me in range(N_DEV):
            diag = (me + 2) % N_DEV
            right = (me + 1) % N_DEV
            left = (me - 1) % N_DEV

            @pl.when(my_i == me)
            def _():
                for d in (diag, right, left):
                    chunk = jnp.dot(x2[d * ROWS:(d + 1) * ROWS, :], w2,
                                    preferred_element_type=jnp.float32)
                    partial_ref[pl.ds(d * ROWS, ROWS), :] = chunk.astype(jnp.bfloat16)
                    rdma = pltpu.make_async_remote_copy(
                        src_ref=partial_ref.at[pl.ds(d * ROWS, ROWS), :],
                        dst_ref=recv_ref.at[me],
                        send_sem=send_sems.at[d],
                        recv_sem=recv_sems.at[me],
                        device_id=(d,),
                        device_id_type=pl.DeviceIdType.MESH,
                    )
                    rdma.start()

                own = jnp.dot(x2[me * ROWS:(me + 1) * ROWS, :], w2,
                              preferred_element_type=jnp.float32)

                acc = own
                for j in (left, right, diag):
                    recv = pltpu.make_async_remote_copy(
                        src_ref=partial_ref.at[pl.ds(0, ROWS), :],
                        dst_ref=recv_ref.at[j],
                        send_sem=send_sems.at[j],
                        recv_sem=recv_sems.at[j],
                        device_id=(j,),
                        device_id_type=pl.DeviceIdType.MESH,
                    )
                    recv.wait_recv()
                    acc = acc + recv_ref[j].astype(jnp.float32)
                out_ref[:, :] = acc

                for d in (diag, right, left):
                    send = pltpu.make_async_remote_copy(
                        src_ref=partial_ref.at[pl.ds(d * ROWS, ROWS), :],
                        dst_ref=recv_ref.at[0],
                        send_sem=send_sems.at[d],
                        recv_sem=recv_sems.at[d],
                        device_id=(d,),
                        device_id_type=pl.DeviceIdType.MESH,
                    )
                    send.wait_send()

    return pl.pallas_call(
        body,
        out_shape=jax.ShapeDtypeStruct((ROWS, D_OUT), jnp.float32),
        in_specs=[
            pl.BlockSpec(memory_space=pltpu.VMEM),
            pl.BlockSpec(memory_space=pltpu.VMEM),
            pl.BlockSpec(memory_space=pltpu.VMEM),
            pl.BlockSpec(memory_space=pltpu.VMEM),
        ],
        out_specs=pl.BlockSpec(memory_space=pltpu.VMEM),
        scratch_shapes=[
            pltpu.VMEM((N_TOK, D_OUT), jnp.bfloat16),
            pltpu.VMEM((N_DEV, ROWS, D_OUT), jnp.bfloat16),
            pltpu.SemaphoreType.DMA((N_DEV,)),
            pltpu.SemaphoreType.DMA((N_DEV,)),
        ],
        compiler_params=pltpu.CompilerParams(collective_id=0),
    )(x, router_W, route_idx, expert_W)
